# baseline (speedup 1.0000x reference)
import functools
import jax
import jax.numpy as jnp
from jax import lax
from jax.experimental import pallas as pl
from jax.experimental.pallas import tpu as pltpu
from jax.experimental.pallas import tpu_sc as plsc

N = 10000
E = 320000
D_IN = 128
H = 16
C = 40

NPAD = 10016
PR = NPAD // 8
NP = 10240
RPT = 640
B = 128
CH = 80
ER = E // B
LW = 31
LR = ER - LW * CH
NC = 2
NS = 16
K = 8

_mesh = plsc.VectorSubcoreMesh(core_axis_name="c", subcore_axis_name="s")


def _stage_raw(e3d1, buf, wid):
    @pl.when(wid < LW)
    def _():
        pltpu.sync_copy(e3d1.at[pl.ds(wid * CH, CH)], buf)

    @pl.when(wid == LW)
    def _():
        pltpu.sync_copy(e3d1.at[pl.ds(LW * CH, LR)], buf.at[pl.ds(0, LR)])

        def fill(r, _):
            for j in range(B // 16):
                buf[r, pl.ds(16 * j, 16)] = jnp.full((16,), N, jnp.int32)
            return 0

        lax.fori_loop(LR, CH, fill, 0)


DB = 1024
DCH = CH * B // DB


def _deg_kernel(e1d, zflat, out, dbuf, onev, ssem, deg_sp):
    cid = lax.axis_index("c")
    sid = lax.axis_index("s")
    wid = cid * NS + sid
    ept = CH * B

    @pl.when(wid < LW)
    def _():
        pltpu.sync_copy(e1d.at[1].at[pl.ds(wid * ept, ept)], dbuf)

    @pl.when(wid == LW)
    def _():
        pltpu.sync_copy(e1d.at[1].at[pl.ds(LW * ept, LR * B)],
                        dbuf.at[pl.ds(0, LR * B)])

        def fill(i, _):
            dbuf[pl.ds(LR * B + 16 * i, 16)] = jnp.full((16,), N, jnp.int32)
            return 0

        lax.fori_loop(0, (CH - LR) * B // 16, fill, 0)

    for j in range(DB // 16):
        onev[pl.ds(16 * j, 16)] = jnp.ones((16,), jnp.float32)
    pltpu.sync_copy(zflat, deg_sp.at[pl.ds(sid * RPT, RPT)])
    plsc.subcore_barrier()

    def fire(c, _):
        pltpu.async_copy(onev, deg_sp.at[dbuf.at[pl.ds(c * DB, DB)]],
                         ssem, add=True)
        return 0

    lax.fori_loop(0, DCH, fire, 0)

    def drain(c, _):
        pltpu.make_async_copy(onev, deg_sp.at[dbuf.at[pl.ds(0, DB)]],
                              ssem).wait()
        return 0

    lax.fori_loop(0, DCH, drain, 0)
    plsc.subcore_barrier()
    pltpu.sync_copy(deg_sp.at[pl.ds(sid * RPT, RPT)],
                    out.at[cid].at[pl.ds(sid * RPT, RPT)])


def _agg_kernel(y, e3d, z2d, out, sbuf, dbuf, *rest):
    rows = rest[:K]
    gs = rest[K:2 * K]
    ss = rest[2 * K:3 * K]
    agg_sp = rest[3 * K]
    y_sp = rest[3 * K + 1]
    cid = lax.axis_index("c")
    sid = lax.axis_index("s")
    wid = cid * NS + sid
    _stage_raw(e3d.at[0], sbuf, wid)
    _stage_raw(e3d.at[1], dbuf, wid)
    pltpu.sync_copy(z2d, agg_sp.at[pl.ds(sid * RPT, RPT)])
    pltpu.sync_copy(y.at[pl.ds(sid * (NPAD // NS), NPAD // NS)],
                    y_sp.at[pl.ds(sid * (NPAD // NS), NPAD // NS)])
    plsc.subcore_barrier()

    for b in range(K - 1):
        pltpu.async_copy(y_sp.at[sbuf.at[b]], rows[b], gs[b])

    def body(ck, _):
        for b in range(K):
            c = K * ck + b
            nb = (b + K - 1) % K
            pltpu.make_async_copy(y_sp.at[sbuf.at[c]], rows[b], gs[b]).wait()
            pltpu.async_copy(rows[b], agg_sp.at[dbuf.at[c]], ss[b], add=True)
            if b == 0:
                @pl.when(ck >= 1)
                def _():
                    pltpu.make_async_copy(
                        rows[nb], agg_sp.at[dbuf.at[0]], ss[nb]).wait()
                pltpu.async_copy(y_sp.at[sbuf.at[c + K - 1]], rows[nb],
                                 gs[nb])
            else:
                @pl.when(ck < CH // K - 1)
                def _():
                    pltpu.make_async_copy(
                        rows[nb], agg_sp.at[dbuf.at[0]], ss[nb]).wait()
                    pltpu.async_copy(y_sp.at[sbuf.at[c + K - 1]], rows[nb],
                                     gs[nb])
        return 0

    lax.fori_loop(0, CH // K, body, 0)
    for b in range(K):
        pltpu.make_async_copy(rows[b], agg_sp.at[dbuf.at[0]], ss[b]).wait()
    plsc.subcore_barrier()
    pltpu.sync_copy(agg_sp.at[pl.ds(sid * RPT, RPT)],
                    out.at[cid].at[pl.ds(sid * RPT, RPT)])


_sc_params = pltpu.CompilerParams(use_tc_tiling_on_sc=False)

_sc_deg = pl.kernel(
    _deg_kernel,
    out_type=jax.ShapeDtypeStruct((NC, NP), jnp.float32),
    mesh=_mesh,
    compiler_params=_sc_params,
    scratch_types=[
        pltpu.VMEM((CH * B,), jnp.int32),
        pltpu.VMEM((DB,), jnp.float32),
        pltpu.SemaphoreType.DMA,
        pltpu.VMEM_SHARED((NP,), jnp.float32),
    ],
)

_sc_agg = pl.kernel(
    _agg_kernel,
    out_type=jax.ShapeDtypeStruct((NC, NP, H), jnp.float32),
    mesh=_mesh,
    compiler_params=_sc_params,
    scratch_types=(
        [pltpu.VMEM((CH, B), jnp.int32)] * 2
        + [pltpu.VMEM((B, H), jnp.float32)] * K
        + [pltpu.SemaphoreType.DMA] * (2 * K)
        + [pltpu.VMEM_SHARED((NP, H), jnp.float32)]
        + [pltpu.VMEM_SHARED((NPAD, H), jnp.float32)]
    ),
)


def _tc_a_body(xp_ref, w1b_ref, dinvw_ref, z1_ref):
    y = jnp.dot(xp_ref[...], w1b_ref[...], preferred_element_type=jnp.float32)
    z1_ref[:N // 8, :] = y * dinvw_ref[:N // 8, :]
    z1_ref[N // 8:, :] = jnp.zeros((PR - N // 8, 128), jnp.float32)


def _tc_b_body(aggp_ref, z1_ref, dinvw_ref, b1w_ref, g1_ref):
    agg = aggp_ref[0, :PR, :] + aggp_ref[1, :PR, :] + z1_ref[...]
    h = jnp.maximum(agg * dinvw_ref[...] + b1w_ref[...], 0.0)
    g1_ref[...] = h * dinvw_ref[...]


def _tc_c_body(aggp_ref, g1_ref, dinvw_ref, w2b_ref, b2w_ref, g2_ref):
    s = aggp_ref[0, :PR, :] + aggp_ref[1, :PR, :] + g1_ref[...]
    t = jnp.dot(s, w2b_ref[...], preferred_element_type=jnp.float32)
    h = jnp.maximum(t * dinvw_ref[...] + b2w_ref[...], 0.0)
    g2_ref[...] = h * dinvw_ref[...]


def _tc_d_body(aggp_ref, g2_ref, dinv40_ref, w3b_ref, b3w_ref, out_ref):
    s = aggp_ref[0, :N // 8, :] + aggp_ref[1, :N // 8, :] + g2_ref[:N // 8, :]
    t = jnp.dot(s, w3b_ref[...], preferred_element_type=jnp.float32)
    logits = t * dinv40_ref[...] + b3w_ref[...]
    for seg in range(8):
        lg = logits[:, seg * C:(seg + 1) * C]
        m = jnp.max(lg, axis=1, keepdims=True)
        e = jnp.exp(lg - m)
        lse = jnp.log(jnp.sum(e, axis=1, keepdims=True)) + m
        out_ref[:, seg * C:(seg + 1) * C] = lg - lse


_tc_a = pl.pallas_call(
    _tc_a_body, out_shape=jax.ShapeDtypeStruct((PR, 128), jnp.float32))
_tc_b = pl.pallas_call(
    _tc_b_body, out_shape=jax.ShapeDtypeStruct((PR, 128), jnp.float32))
_tc_c = pl.pallas_call(
    _tc_c_body, out_shape=jax.ShapeDtypeStruct((PR, 128), jnp.float32))
_tc_d = pl.pallas_call(
    _tc_d_body, out_shape=jax.ShapeDtypeStruct((N // 8, 8 * C), jnp.float32))


@jax.jit
def kernel(x, edge_index, W1, b1, W2, b2, W3, b3):
    if edge_index.dtype != jnp.int32:
        edge_index = edge_index.astype(jnp.int32)
    e3d = edge_index.reshape(2, ER, B)
    xp = x.reshape(N // 8, 8 * D_IN)
    zflat = jnp.zeros((RPT,), jnp.float32)
    z2d = jnp.zeros((RPT, H), jnp.float32)
    eye8 = jnp.eye(8, dtype=jnp.float32)
    w1b = jnp.kron(eye8, W1)
    w2b = jnp.kron(eye8, W2)
    w3b = jnp.kron(eye8, W3)
    b1w = jnp.tile(b1, 8).reshape(1, 128)
    b2w = jnp.tile(b2, 8).reshape(1, 128)
    b3w = jnp.tile(b3, 8).reshape(1, 8 * C)

    degp = _sc_deg(edge_index, zflat)
    deg = degp[0, :NPAD] + degp[1, :NPAD] + 1.0
    dinv = lax.rsqrt(deg)
    dinvw = jnp.repeat(dinv, H).reshape(PR, 128)
    dinv40 = jnp.repeat(dinv[:N], C).reshape(N // 8, 8 * C)

    z1p = _tc_a(xp, w1b, dinvw)
    aggp1 = _sc_agg(z1p.reshape(NPAD, H), e3d, z2d)
    g1p = _tc_b(aggp1.reshape(NC, NP // 8, 128), z1p, dinvw, b1w)
    aggp2 = _sc_agg(g1p.reshape(NPAD, H), e3d, z2d)
    g2p = _tc_c(aggp2.reshape(NC, NP // 8, 128), g1p, dinvw, w2b, b2w)
    aggp3 = _sc_agg(g2p.reshape(NPAD, H), e3d, z2d)
    outp = _tc_d(aggp3.reshape(NC, NP // 8, 128), g2p, dinv40, w3b, b3w)
    return outp.reshape(N, C)


# speedup vs baseline: 1.0120x; 1.0120x over previous
import functools
import jax
import jax.numpy as jnp
from jax import lax
from jax.experimental import pallas as pl
from jax.experimental.pallas import tpu as pltpu
from jax.experimental.pallas import tpu_sc as plsc

N = 10000
E = 320000
D_IN = 128
H = 16
C = 40

NPAD = 10016
PR = NPAD // 8
NP = 10240
RPT = 640
B = 128
CH = 80
ER = E // B
LW = 31
LR = ER - LW * CH
NC = 2
NS = 16
K = 8

_mesh = plsc.VectorSubcoreMesh(core_axis_name="c", subcore_axis_name="s")


def _stage_raw(e3d1, buf, wid):
    @pl.when(wid < LW)
    def _():
        pltpu.sync_copy(e3d1.at[pl.ds(wid * CH, CH)], buf)

    @pl.when(wid == LW)
    def _():
        pltpu.sync_copy(e3d1.at[pl.ds(LW * CH, LR)], buf.at[pl.ds(0, LR)])

        def fill(r, _):
            for j in range(B // 16):
                buf[r, pl.ds(16 * j, 16)] = jnp.full((16,), N, jnp.int32)
            return 0

        lax.fori_loop(LR, CH, fill, 0)


def _deg_kernel(e3d, zflat, out, dbuf, onev, ssem, deg_sp):
    cid = lax.axis_index("c")
    sid = lax.axis_index("s")
    wid = cid * NS + sid
    _stage_raw(e3d.at[1], dbuf, wid)
    for j in range(B // 16):
        onev[pl.ds(16 * j, 16)] = jnp.ones((16,), jnp.float32)
    pltpu.sync_copy(zflat, deg_sp.at[pl.ds(sid * RPT, RPT)])
    plsc.subcore_barrier()

    def fire(c, _):
        pltpu.async_copy(onev, deg_sp.at[dbuf.at[c]], ssem, add=True)
        return 0

    lax.fori_loop(0, CH, fire, 0)

    def drain(c, _):
        pltpu.make_async_copy(onev, deg_sp.at[dbuf.at[0]], ssem).wait()
        return 0

    lax.fori_loop(0, CH, drain, 0)
    plsc.subcore_barrier()
    pltpu.sync_copy(deg_sp.at[pl.ds(sid * RPT, RPT)],
                    out.at[cid].at[pl.ds(sid * RPT, RPT)])


def _agg_kernel(y, e3d, z2d, out, sbuf, dbuf, *rest):
    rows = rest[:K]
    gs = rest[K:2 * K]
    ss = rest[2 * K:3 * K]
    agg_sp = rest[3 * K]
    y_sp = rest[3 * K + 1]
    cid = lax.axis_index("c")
    sid = lax.axis_index("s")
    wid = cid * NS + sid
    _stage_raw(e3d.at[0], sbuf, wid)
    _stage_raw(e3d.at[1], dbuf, wid)
    pltpu.sync_copy(z2d, agg_sp.at[pl.ds(sid * RPT, RPT)])
    pltpu.sync_copy(y.at[pl.ds(sid * (NPAD // NS), NPAD // NS)],
                    y_sp.at[pl.ds(sid * (NPAD // NS), NPAD // NS)])
    plsc.subcore_barrier()

    for b in range(K - 1):
        pltpu.async_copy(y_sp.at[sbuf.at[b]], rows[b], gs[b])

    def body(ck, _):
        for b in range(K):
            c = K * ck + b
            nb = (b + K - 1) % K
            pltpu.make_async_copy(y_sp.at[sbuf.at[c]], rows[b], gs[b]).wait()
            pltpu.async_copy(rows[b], agg_sp.at[dbuf.at[c]], ss[b], add=True)
            if b == 0:
                @pl.when(ck >= 1)
                def _():
                    pltpu.make_async_copy(
                        rows[nb], agg_sp.at[dbuf.at[0]], ss[nb]).wait()
                pltpu.async_copy(y_sp.at[sbuf.at[c + K - 1]], rows[nb],
                                 gs[nb])
            else:
                @pl.when(ck < CH // K - 1)
                def _():
                    pltpu.make_async_copy(
                        rows[nb], agg_sp.at[dbuf.at[0]], ss[nb]).wait()
                    pltpu.async_copy(y_sp.at[sbuf.at[c + K - 1]], rows[nb],
                                     gs[nb])
        return 0

    lax.fori_loop(0, CH // K, body, 0)
    for b in range(K):
        pltpu.make_async_copy(rows[b], agg_sp.at[dbuf.at[0]], ss[b]).wait()
    plsc.subcore_barrier()
    pltpu.sync_copy(agg_sp.at[pl.ds(sid * RPT, RPT)],
                    out.at[cid].at[pl.ds(sid * RPT, RPT)])


_sc_params = pltpu.CompilerParams(use_tc_tiling_on_sc=False)

_sc_deg = pl.kernel(
    _deg_kernel,
    out_type=jax.ShapeDtypeStruct((NC, NP), jnp.float32),
    mesh=_mesh,
    compiler_params=_sc_params,
    scratch_types=[
        pltpu.VMEM((CH, B), jnp.int32),
        pltpu.VMEM((B,), jnp.float32),
        pltpu.SemaphoreType.DMA,
        pltpu.VMEM_SHARED((NP,), jnp.float32),
    ],
)

_sc_agg = pl.kernel(
    _agg_kernel,
    out_type=jax.ShapeDtypeStruct((NC, NP, H), jnp.float32),
    mesh=_mesh,
    compiler_params=_sc_params,
    scratch_types=(
        [pltpu.VMEM((CH, B), jnp.int32)] * 2
        + [pltpu.VMEM((B, H), jnp.float32)] * K
        + [pltpu.SemaphoreType.DMA] * (2 * K)
        + [pltpu.VMEM_SHARED((NP, H), jnp.float32)]
        + [pltpu.VMEM_SHARED((NPAD, H), jnp.float32)]
    ),
)


def _tc_a_body(xp_ref, w1b_ref, dinvw_ref, z1_ref):
    y = jnp.dot(xp_ref[...], w1b_ref[...], preferred_element_type=jnp.float32)
    z1_ref[:N // 8, :] = y * dinvw_ref[:N // 8, :]
    z1_ref[N // 8:, :] = jnp.zeros((PR - N // 8, 128), jnp.float32)


def _tc_b_body(aggp_ref, z1_ref, dinvw_ref, b1w_ref, g1_ref):
    agg = aggp_ref[0, :PR, :] + aggp_ref[1, :PR, :] + z1_ref[...]
    h = jnp.maximum(agg * dinvw_ref[...] + b1w_ref[...], 0.0)
    g1_ref[...] = h * dinvw_ref[...]


def _tc_c_body(aggp_ref, g1_ref, dinvw_ref, w2b_ref, b2w_ref, g2_ref):
    s = aggp_ref[0, :PR, :] + aggp_ref[1, :PR, :] + g1_ref[...]
    t = jnp.dot(s, w2b_ref[...], preferred_element_type=jnp.float32)
    h = jnp.maximum(t * dinvw_ref[...] + b2w_ref[...], 0.0)
    g2_ref[...] = h * dinvw_ref[...]


def _tc_d_body(aggp_ref, g2_ref, dinv40_ref, w3b_ref, b3w_ref, out_ref):
    s = aggp_ref[0, :N // 8, :] + aggp_ref[1, :N // 8, :] + g2_ref[:N // 8, :]
    t = jnp.dot(s, w3b_ref[...], preferred_element_type=jnp.float32)
    logits = t * dinv40_ref[...] + b3w_ref[...]
    for seg in range(8):
        lg = logits[:, seg * C:(seg + 1) * C]
        m = jnp.max(lg, axis=1, keepdims=True)
        e = jnp.exp(lg - m)
        lse = jnp.log(jnp.sum(e, axis=1, keepdims=True)) + m
        out_ref[:, seg * C:(seg + 1) * C] = lg - lse


_tc_a = pl.pallas_call(
    _tc_a_body, out_shape=jax.ShapeDtypeStruct((PR, 128), jnp.float32))
_tc_b = pl.pallas_call(
    _tc_b_body, out_shape=jax.ShapeDtypeStruct((PR, 128), jnp.float32))
_tc_c = pl.pallas_call(
    _tc_c_body, out_shape=jax.ShapeDtypeStruct((PR, 128), jnp.float32))
_tc_d = pl.pallas_call(
    _tc_d_body, out_shape=jax.ShapeDtypeStruct((N // 8, 8 * C), jnp.float32))


@jax.jit
def kernel(x, edge_index, W1, b1, W2, b2, W3, b3):
    if edge_index.dtype != jnp.int32:
        edge_index = edge_index.astype(jnp.int32)
    e3d = edge_index.reshape(2, ER, B)
    xp = x.reshape(N // 8, 8 * D_IN)
    zflat = jnp.zeros((RPT,), jnp.float32)
    z2d = jnp.zeros((RPT, H), jnp.float32)
    eye8 = jnp.eye(8, dtype=jnp.float32)
    w1b = jnp.kron(eye8, W1)
    w2b = jnp.kron(eye8, W2)
    w3b = jnp.kron(eye8, W3)
    b1w = jnp.tile(b1, 8).reshape(1, 128)
    b2w = jnp.tile(b2, 8).reshape(1, 128)
    b3w = jnp.tile(b3, 8).reshape(1, 8 * C)

    degp = _sc_deg(e3d, zflat)
    deg = degp[0, :NPAD] + degp[1, :NPAD] + 1.0
    dinv = lax.rsqrt(deg)
    dinvw = jnp.repeat(dinv, H).reshape(PR, 128)
    dinv40 = jnp.repeat(dinv[:N], C).reshape(N // 8, 8 * C)

    z1p = _tc_a(xp, w1b, dinvw)
    aggp1 = _sc_agg(z1p.reshape(NPAD, H), e3d, z2d)
    g1p = _tc_b(aggp1.reshape(NC, NP // 8, 128), z1p, dinvw, b1w)
    aggp2 = _sc_agg(g1p.reshape(NPAD, H), e3d, z2d)
    g2p = _tc_c(aggp2.reshape(NC, NP // 8, 128), g1p, dinvw, w2b, b2w)
    aggp3 = _sc_agg(g2p.reshape(NPAD, H), e3d, z2d)
    outp = _tc_d(aggp3.reshape(NC, NP // 8, 128), g2p, dinv40, w3b, b3w)
    return outp.reshape(N, C)
